# Initial kernel scaffold; baseline (speedup 1.0000x reference)
import jax
import jax.numpy as jnp
from jax.experimental import pallas as pl


def kernel(x, edge_index, W1, b1, g1, beta1, W2, b2, g2, beta2):
    raise NotImplementedError("write your pallas kernel here")


# speedup vs baseline: 4.6000x; 4.6000x over previous
import functools

import jax
import jax.numpy as jnp
from jax import lax
from jax.experimental import pallas as pl
from jax.experimental.pallas import tpu as pltpu
from jax.experimental.pallas import tpu_sc as plsc

N = 10000
D = 128
E = 320000
EPS_GIN = 128.0
BN_EPS = 1e-5

NC = 2
NS = 16
NW = NC * NS
EDGE_BLK = 128
BLKS_PER_TILE = -(-E // (NW * EDGE_BLK))
E_PER_TILE = BLKS_PER_TILE * EDGE_BLK
E_PAD = E_PER_TILE * NW
ACC_ROWS_PER_TILE = ((-(-(N + 1) // NS) + 7) // 8) * 8
ACC_ROWS = ACC_ROWS_PER_TILE * NS


def _sc_agg_body(x_hbm, src_hbm, dst_hbm, z_hbm, out_hbm,
                 acc_sh, src_v, dst_v, rows_v, sem):
    c = lax.axis_index("c")
    s = lax.axis_index("s")
    wid = c * NS + s

    pltpu.sync_copy(z_hbm, acc_sh.at[pl.ds(s * ACC_ROWS_PER_TILE,
                                           ACC_ROWS_PER_TILE)])
    pltpu.sync_copy(src_hbm.at[wid], src_v)
    pltpu.sync_copy(dst_hbm.at[wid], dst_v)
    plsc.subcore_barrier()

    def step(j, carry):
        pltpu.async_copy(x_hbm.at[src_v.at[j]], rows_v, sem).wait()
        pltpu.sync_copy(rows_v, acc_sh.at[dst_v.at[j]], add=True)
        return carry

    lax.fori_loop(0, BLKS_PER_TILE, step, 0)
    plsc.subcore_barrier()

    pltpu.sync_copy(acc_sh.at[pl.ds(s * ACC_ROWS_PER_TILE,
                                    ACC_ROWS_PER_TILE)],
                    out_hbm.at[c, pl.ds(s * ACC_ROWS_PER_TILE,
                                        ACC_ROWS_PER_TILE)])


_sc_agg = functools.partial(
    pl.kernel,
    out_type=jax.ShapeDtypeStruct((NC, ACC_ROWS, D), jnp.float32),
    mesh=plsc.VectorSubcoreMesh(core_axis_name="c", subcore_axis_name="s"),
    scratch_types=[
        pltpu.VMEM_SHARED((ACC_ROWS, D), jnp.float32),
        pltpu.VMEM((BLKS_PER_TILE, EDGE_BLK), jnp.int32),
        pltpu.VMEM((BLKS_PER_TILE, EDGE_BLK), jnp.int32),
        pltpu.VMEM((EDGE_BLK, D), jnp.float32),
        pltpu.SemaphoreType.DMA,
    ],
)(_sc_agg_body)


def _mlp_body(x_ref, p_ref, w1t_ref, b1_ref, g1_ref, bt1_ref,
              w2t_ref, b2_ref, g2_ref, bt2_ref, o_ref):
    p = p_ref[...]
    h = x_ref[...] * (1.0 + EPS_GIN) + p[0, :N] + p[1, :N]
    h = jnp.dot(h, w1t_ref[...], preferred_element_type=jnp.float32)
    h = jnp.maximum(h + b1_ref[...], 0.0)
    mean = jnp.mean(h, axis=0, keepdims=True)
    var = jnp.mean((h - mean) ** 2, axis=0, keepdims=True)
    h = (h - mean) / jnp.sqrt(var + BN_EPS) * g1_ref[...] + bt1_ref[...]
    h = jnp.dot(h, w2t_ref[...], preferred_element_type=jnp.float32)
    h = jnp.maximum(h + b2_ref[...], 0.0)
    mean = jnp.mean(h, axis=0, keepdims=True)
    var = jnp.mean((h - mean) ** 2, axis=0, keepdims=True)
    o_ref[...] = (h - mean) / jnp.sqrt(var + BN_EPS) * g2_ref[...] + bt2_ref[...]


_mlp = pl.pallas_call(
    _mlp_body,
    out_shape=jax.ShapeDtypeStruct((N, D), jnp.float32),
)


@jax.jit
def kernel(x, edge_index, W1, b1, g1, beta1, W2, b2, g2, beta2):
    ei = edge_index.astype(jnp.int32)
    pad = E_PAD - E
    src = jnp.concatenate([ei[0], jnp.zeros((pad,), jnp.int32)])
    dst = jnp.concatenate([ei[1], jnp.full((pad,), N, jnp.int32)])
    src = src.reshape(NW, BLKS_PER_TILE, EDGE_BLK)
    dst = dst.reshape(NW, BLKS_PER_TILE, EDGE_BLK)
    z = jnp.zeros((ACC_ROWS_PER_TILE, D), jnp.float32)
    parts = _sc_agg(x, src, dst, z)
    return _mlp(x, parts,
                W1.T, b1.reshape(1, D), g1.reshape(1, D), beta1.reshape(1, D),
                W2.T, b2.reshape(1, D), g2.reshape(1, D), beta2.reshape(1, D))
